# baseline (speedup 1.0000x reference)
import functools

import jax
import jax.numpy as jnp
from jax.experimental import pallas as pl
from jax.experimental.pallas import tpu as pltpu
from jax.experimental.pallas import tpu_sc as plsc

_NUM_BASKETS = 4000
_N_NODES = 10000
_N_NODES_PAD = 10240
_D = 128
_NI = 4
_H = 128
_B = 256
_L = 50
_NC = 2
_NS = 16
_NW = _NC * _NS
_EDGE_CHUNK = 128
_ROWS_PER_TILE = _N_NODES_PAD // _NS


def _gcn_layer_sc(emb, packed, wch, zeros_tile, n_chunks):
  mesh = plsc.VectorSubcoreMesh(core_axis_name="c", subcore_axis_name="s")

  n_total = 2 * n_chunks
  n_fast = (n_total * 16 // 16) & ~1
  n_slow = n_total - n_fast

  def body(emb_hbm, pk_hbm, w_hbm, z_hbm, out_hbm,
           eb0, eb1, wb0, wb1, rows0, rows1, accum,
           gsem0, gsem1, ssem0, ssem1):
    cid = jax.lax.axis_index("c")
    sid = jax.lax.axis_index("s")
    ebs = (eb0, eb1)
    wbs = (wb0, wb1)
    rws = (rows0, rows1)
    gs = (gsem0, gsem1)
    ss = (ssem0, ssem1)
    tile_sl = pl.ds(sid * _ROWS_PER_TILE, _ROWS_PER_TILE)
    pltpu.sync_copy(z_hbm, accum.at[tile_sl])
    plsc.subcore_barrier()

    my_n = jnp.where(cid == 0, n_fast, n_slow)
    cbase = jnp.where(cid == 0, sid * n_fast,
                      _NS * n_fast + sid * n_slow)
    @pl.when(my_n > 0)
    def _prime():
      pltpu.sync_copy(pk_hbm.at[cbase], eb0)
      pltpu.sync_copy(w_hbm.at[cbase], wb0)
      pltpu.async_copy(emb_hbm.at[eb0.at[0]], rows0, gsem0)

    @pl.loop(0, my_n, step=2)
    def _chunk(jj):
      for b in range(2):
        j = jj + b
        cur, nxt = b, 1 - b

        @pl.when(j + 1 < my_n)
        def _prefetch():
          @pl.when(j >= 1)
          def _():
            pltpu.make_async_copy(
                rws[nxt], accum.at[ebs[nxt].at[1]], ss[nxt]).wait()
          pltpu.sync_copy(pk_hbm.at[cbase + j + 1], ebs[nxt])
          pltpu.sync_copy(w_hbm.at[cbase + j + 1], wbs[nxt])
          pltpu.async_copy(emb_hbm.at[ebs[nxt].at[0]], rws[nxt], gs[nxt])

        pltpu.make_async_copy(
            emb_hbm.at[ebs[cur].at[0]], rws[cur], gs[cur]).wait()

        @pl.loop(0, _EDGE_CHUNK // 16)
        def _scale(q):
          wvec = wbs[cur][pl.ds(q * 16, 16)]
          for l in range(16):
            we = wvec[l]
            e = q * 16 + l
            for g in range(_D // 16):
              sl = pl.ds(g * 16, 16)
              rws[cur][e, sl] = rws[cur][e, sl] * we

        pltpu.async_copy(rws[cur], accum.at[ebs[cur].at[1]], ss[cur],
                         add=True)

    @pl.when(my_n > 0)
    def _drain():
      for b in range(2):
        pltpu.make_async_copy(rws[b], accum.at[ebs[b].at[1]], ss[b]).wait()

    plsc.subcore_barrier()
    pltpu.sync_copy(accum.at[tile_sl], out_hbm.at[cid, tile_sl])

  k = pl.kernel(
      body,
      out_type=jax.ShapeDtypeStruct((_NC, _N_NODES_PAD, _D), jnp.float32),
      mesh=mesh,
      scratch_types=[
          pltpu.VMEM((2, _EDGE_CHUNK), jnp.int32),
          pltpu.VMEM((2, _EDGE_CHUNK), jnp.int32),
          pltpu.VMEM((_EDGE_CHUNK,), jnp.float32),
          pltpu.VMEM((_EDGE_CHUNK,), jnp.float32),
          pltpu.VMEM((_EDGE_CHUNK, _D), jnp.float32),
          pltpu.VMEM((_EDGE_CHUNK, _D), jnp.float32),
          pltpu.VMEM_SHARED((_N_NODES_PAD, _D), jnp.float32),
          pltpu.SemaphoreType.DMA,
          pltpu.SemaphoreType.DMA,
          pltpu.SemaphoreType.DMA,
          pltpu.SemaphoreType.DMA,
      ],
  )
  return k(emb, packed, wch, zeros_tile)


_CBLK = 1000


def _combine_mid(p, acc):
  def body(p_ref, a_ref, e_ref, ao_ref):
    e = p_ref[0] + p_ref[1]
    e_ref[...] = e
    ao_ref[...] = a_ref[...] + e

  return pl.pallas_call(
      body,
      grid=(_N_NODES // _CBLK,),
      in_specs=[
          pl.BlockSpec((2, _CBLK, _D), lambda i: (0, i, 0)),
          pl.BlockSpec((_CBLK, _D), lambda i: (i, 0)),
      ],
      out_specs=[
          pl.BlockSpec((_CBLK, _D), lambda i: (i, 0)),
          pl.BlockSpec((_CBLK, _D), lambda i: (i, 0)),
      ],
      out_shape=[
          jax.ShapeDtypeStruct((_N_NODES, _D), jnp.float32),
          jax.ShapeDtypeStruct((_N_NODES, _D), jnp.float32),
      ],
  )(p, acc)


def _combine_final(p, acc):
  def body(p_ref, a_ref, o_ref):
    o_ref[...] = (a_ref[...] + p_ref[0] + p_ref[1]) * 0.25

  return pl.pallas_call(
      body,
      grid=(_N_NODES // _CBLK,),
      in_specs=[
          pl.BlockSpec((2, _CBLK, _D), lambda i: (0, i, 0)),
          pl.BlockSpec((_CBLK, _D), lambda i: (i, 0)),
      ],
      out_specs=pl.BlockSpec((_CBLK, _D), lambda i: (i, 0)),
      out_shape=jax.ShapeDtypeStruct((_N_NODES, _D), jnp.float32),
  )(p, acc)


def _gather_sc(table, idx):
  n = idx.shape[0]
  per_w = n // _NW
  ch = 80
  nch = per_w // ch
  mesh = plsc.VectorSubcoreMesh(core_axis_name="c", subcore_axis_name="s")

  def body(t_hbm, i_hbm, o_hbm, iv, rows, sem):
    cid = jax.lax.axis_index("c")
    sid = jax.lax.axis_index("s")
    wid = sid * _NC + cid
    base = wid * per_w

    @pl.loop(0, nch)
    def _(j):
      off = base + j * ch
      pltpu.sync_copy(i_hbm.at[pl.ds(off, ch)], iv)
      pltpu.async_copy(t_hbm.at[iv], rows, sem).wait()
      pltpu.sync_copy(rows, o_hbm.at[pl.ds(off, ch)])

  k = pl.kernel(
      body,
      out_type=jax.ShapeDtypeStruct((n, _D), jnp.float32),
      mesh=mesh,
      scratch_types=[
          pltpu.VMEM((ch,), jnp.int32),
          pltpu.VMEM((ch, _D), jnp.float32),
          pltpu.SemaphoreType.DMA,
      ],
  )
  return k(table, idx)


def _gru_tc(bemb, Wt, wih0T, whh0T, bih0, bhh0, wih1T, whh1T, bih1, bhh1,
            blen4):
  nb4 = _NI * _B

  def body(e_ref, wt_ref, wih0_ref, whh0_ref, bih0_ref, bhh0_ref,
           wih1_ref, whh1_ref, bih1_ref, bhh1_ref, blen_ref, out_ref,
           h1_s, h2_s, x_s):
    t = pl.program_id(0)

    @pl.when(t == 0)
    def _():
      h1_s[...] = jnp.zeros_like(h1_s)
      h2_s[...] = jnp.zeros_like(h2_s)

    e = e_ref[0]
    for i in range(_NI):
      x_s[pl.ds(i * _B, _B), :] = jnp.dot(
          e, wt_ref[i], preferred_element_type=jnp.float32)
    x = x_s[...]

    h1p = h1_s[...]
    gi = jnp.dot(x, wih0_ref[...],
                 preferred_element_type=jnp.float32) + bih0_ref[...]
    gh = jnp.dot(h1p, whh0_ref[...],
                 preferred_element_type=jnp.float32) + bhh0_ref[...]
    r = jax.nn.sigmoid(gi[:, :_H] + gh[:, :_H])
    z = jax.nn.sigmoid(gi[:, _H:2 * _H] + gh[:, _H:2 * _H])
    n = jnp.tanh(gi[:, 2 * _H:] + r * gh[:, 2 * _H:])
    h1 = (1.0 - z) * n + z * h1p
    h1_s[...] = h1

    h2p = h2_s[...]
    gi2 = jnp.dot(h1, wih1_ref[...],
                  preferred_element_type=jnp.float32) + bih1_ref[...]
    gh2 = jnp.dot(h2p, whh1_ref[...],
                  preferred_element_type=jnp.float32) + bhh1_ref[...]
    r2 = jax.nn.sigmoid(gi2[:, :_H] + gh2[:, :_H])
    z2 = jax.nn.sigmoid(gi2[:, _H:2 * _H] + gh2[:, _H:2 * _H])
    n2 = jnp.tanh(gi2[:, 2 * _H:] + r2 * gh2[:, 2 * _H:])
    h2 = (1.0 - z2) * n2 + z2 * h2p
    h2_s[...] = h2

    sel = blen_ref[...] == t
    prev = jnp.where(t == 0, jnp.zeros_like(h2), out_ref[...])
    out_ref[...] = jnp.where(sel, h2, prev)

  return pl.pallas_call(
      body,
      grid=(_L,),
      in_specs=[
          pl.BlockSpec((1, _B, _D), lambda t: (t, 0, 0)),
          pl.BlockSpec((_NI, _D, _D), lambda t: (0, 0, 0)),
          pl.BlockSpec((_D, 3 * _H), lambda t: (0, 0)),
          pl.BlockSpec((_H, 3 * _H), lambda t: (0, 0)),
          pl.BlockSpec((1, 3 * _H), lambda t: (0, 0)),
          pl.BlockSpec((1, 3 * _H), lambda t: (0, 0)),
          pl.BlockSpec((_H, 3 * _H), lambda t: (0, 0)),
          pl.BlockSpec((_H, 3 * _H), lambda t: (0, 0)),
          pl.BlockSpec((1, 3 * _H), lambda t: (0, 0)),
          pl.BlockSpec((1, 3 * _H), lambda t: (0, 0)),
          pl.BlockSpec((nb4, _D), lambda t: (0, 0)),
      ],
      out_specs=pl.BlockSpec((nb4, _H), lambda t: (0, 0)),
      out_shape=jax.ShapeDtypeStruct((nb4, _H), jnp.float32),
      scratch_shapes=[
          pltpu.VMEM((nb4, _H), jnp.float32),
          pltpu.VMEM((nb4, _H), jnp.float32),
          pltpu.VMEM((nb4, _D), jnp.float32),
      ],
  )(bemb, Wt, wih0T, whh0T, bih0, bhh0, wih1T, whh1T, bih1, bhh1, blen4)


def _final_tc(E, items, merge_w):
  n_items = items.shape[0]

  def body(e_ref, it_ref, mw_ref, o_ref):
    Ei = [e_ref[pl.ds(i * _B, _B), :] for i in range(_NI)]
    g = [[jnp.sum(Ei[i] * Ei[j], axis=1, keepdims=True)
          for j in range(_NI)] for i in range(_NI)]
    c = None
    for i in range(_NI):
      m = jnp.maximum(jnp.maximum(g[i][0], g[i][1]),
                      jnp.maximum(g[i][2], g[i][3]))
      ex = [jnp.exp(g[i][j] - m) for j in range(_NI)]
      s = ex[0] + ex[1] + ex[2] + ex[3]
      num = (ex[0] * mw_ref[0, 0] + ex[1] * mw_ref[0, 1]
             + ex[2] * mw_ref[0, 2] + ex[3] * mw_ref[0, 3])
      ui = num / s
      c = ui * Ei[i] if c is None else c + ui * Ei[i]
    o_ref[...] = jax.lax.dot_general(
        c, it_ref[...], (((1,), (1,)), ((), ())),
        preferred_element_type=jnp.float32)

  return pl.pallas_call(
      body,
      grid=(1,),
      in_specs=[
          pl.BlockSpec((_NI * _B, _H), lambda i: (0, 0)),
          pl.BlockSpec((n_items, _D), lambda i: (0, 0)),
          pl.BlockSpec(memory_space=pltpu.SMEM),
      ],
      out_specs=pl.BlockSpec((_B, n_items), lambda i: (0, 0)),
      out_shape=jax.ShapeDtypeStruct((_B, n_items), jnp.float32),
  )(E, items, merge_w)


def kernel(bseq, bseq_len, edge_index, edge_weight, emb_basket, emb_item,
           W_bseq, merge_W, Wih0, Whh0, bih0, bhh0, Wih1, Whh1, bih1, bhh1):
  f32 = jnp.float32
  e0 = jnp.concatenate([emb_basket, emb_item], axis=0)

  src = edge_index[0].astype(jnp.int32)
  dst = edge_index[1].astype(jnp.int32)
  w = edge_weight.astype(f32)
  ne = src.shape[0]
  gran = 2 * _EDGE_CHUNK
  per_w = ((ne + _NW - 1) // _NW + gran - 1) // gran * gran
  pad = per_w * _NW - ne
  src_p = jnp.concatenate([src, jnp.zeros((pad,), jnp.int32)])
  dst_p = jnp.concatenate([dst, jnp.zeros((pad,), jnp.int32)])
  w_p = jnp.concatenate([w, jnp.zeros((pad,), f32)])
  n_chunks = per_w // _EDGE_CHUNK
  packed = jnp.stack([
      src_p.reshape(_NW * n_chunks, _EDGE_CHUNK),
      dst_p.reshape(_NW * n_chunks, _EDGE_CHUNK),
  ], axis=1)
  wch = w_p.reshape(_NW * n_chunks, _EDGE_CHUNK)
  zeros_tile = jnp.zeros((_ROWS_PER_TILE, _D), f32)

  emb = e0
  acc = e0
  light = None
  for layer in range(3):
    p = _gcn_layer_sc(emb, packed, wch, zeros_tile, n_chunks)
    if layer < 2:
      emb, acc = _combine_mid(p, acc)
    else:
      light = _combine_final(p, acc)

  idx_t = jnp.transpose(bseq).reshape(-1).astype(jnp.int32)
  bemb = _gather_sc(light, idx_t).reshape(_L, _B, _D)

  Wt = jnp.transpose(W_bseq.reshape(_D, _NI, _D), (1, 2, 0))

  blen = jnp.tile(bseq_len.astype(jnp.int32) - 1, _NI)
  blen4 = jnp.broadcast_to(blen[:, None], (_NI * _B, _H))

  E = _gru_tc(bemb, Wt,
              jnp.transpose(Wih0), jnp.transpose(Whh0),
              bih0.reshape(1, -1), bhh0.reshape(1, -1),
              jnp.transpose(Wih1), jnp.transpose(Whh1),
              bih1.reshape(1, -1), bhh1.reshape(1, -1),
              blen4)

  items = light[_NUM_BASKETS:]
  return _final_tc(E, items, merge_W.astype(f32))


# speedup vs baseline: 1.4919x; 1.4919x over previous
import functools

import jax
import jax.numpy as jnp
from jax.experimental import pallas as pl
from jax.experimental.pallas import tpu as pltpu
from jax.experimental.pallas import tpu_sc as plsc

_NUM_BASKETS = 4000
_N_NODES = 10000
_N_NODES_PAD = 10240
_D = 128
_NI = 4
_H = 128
_B = 256
_L = 50
_NC = 2
_NS = 16
_NW = _NC * _NS
_EDGE_CHUNK = 128
_ROWS_PER_TILE = _N_NODES_PAD // _NS


def _gcn_layer_sc(emb, packed, wch, zeros_tile, n_chunks):
  mesh = plsc.VectorSubcoreMesh(core_axis_name="c", subcore_axis_name="s")

  n_total = 2 * n_chunks
  n_fast = (n_total * 13 // 16) & ~1
  n_slow = n_total - n_fast

  def body(emb_hbm, pk_hbm, w_hbm, z_hbm, out_hbm,
           eb0, eb1, wb0, wb1, rows0, rows1, accum,
           gsem0, gsem1, ssem0, ssem1):
    cid = jax.lax.axis_index("c")
    sid = jax.lax.axis_index("s")
    ebs = (eb0, eb1)
    wbs = (wb0, wb1)
    rws = (rows0, rows1)
    gs = (gsem0, gsem1)
    ss = (ssem0, ssem1)
    tile_sl = pl.ds(sid * _ROWS_PER_TILE, _ROWS_PER_TILE)
    pltpu.sync_copy(z_hbm, accum.at[tile_sl])
    plsc.subcore_barrier()

    my_n = jnp.where(cid == 0, n_fast, n_slow)
    cbase = jnp.where(cid == 0, sid * n_fast,
                      _NS * n_fast + sid * n_slow)
    @pl.when(my_n > 0)
    def _prime():
      pltpu.sync_copy(pk_hbm.at[cbase], eb0)
      pltpu.sync_copy(w_hbm.at[cbase], wb0)
      pltpu.async_copy(emb_hbm.at[eb0.at[0]], rows0, gsem0)

    @pl.loop(0, my_n, step=2)
    def _chunk(jj):
      for b in range(2):
        j = jj + b
        cur, nxt = b, 1 - b

        @pl.when(j + 1 < my_n)
        def _prefetch():
          @pl.when(j >= 1)
          def _():
            pltpu.make_async_copy(
                rws[nxt], accum.at[ebs[nxt].at[1]], ss[nxt]).wait()
          pltpu.sync_copy(pk_hbm.at[cbase + j + 1], ebs[nxt])
          pltpu.sync_copy(w_hbm.at[cbase + j + 1], wbs[nxt])
          pltpu.async_copy(emb_hbm.at[ebs[nxt].at[0]], rws[nxt], gs[nxt])

        pltpu.make_async_copy(
            emb_hbm.at[ebs[cur].at[0]], rws[cur], gs[cur]).wait()

        @pl.loop(0, _EDGE_CHUNK // 16)
        def _scale(q):
          wvec = wbs[cur][pl.ds(q * 16, 16)]
          for l in range(16):
            we = wvec[l]
            e = q * 16 + l
            for g in range(_D // 16):
              sl = pl.ds(g * 16, 16)
              rws[cur][e, sl] = rws[cur][e, sl] * we

        pltpu.async_copy(rws[cur], accum.at[ebs[cur].at[1]], ss[cur],
                         add=True)

    @pl.when(my_n > 0)
    def _drain():
      for b in range(2):
        pltpu.make_async_copy(rws[b], accum.at[ebs[b].at[1]], ss[b]).wait()

    plsc.subcore_barrier()
    pltpu.sync_copy(accum.at[tile_sl], out_hbm.at[cid, tile_sl])

  k = pl.kernel(
      body,
      out_type=jax.ShapeDtypeStruct((_NC, _N_NODES_PAD, _D), jnp.float32),
      mesh=mesh,
      scratch_types=[
          pltpu.VMEM((2, _EDGE_CHUNK), jnp.int32),
          pltpu.VMEM((2, _EDGE_CHUNK), jnp.int32),
          pltpu.VMEM((_EDGE_CHUNK,), jnp.float32),
          pltpu.VMEM((_EDGE_CHUNK,), jnp.float32),
          pltpu.VMEM((_EDGE_CHUNK, _D), jnp.float32),
          pltpu.VMEM((_EDGE_CHUNK, _D), jnp.float32),
          pltpu.VMEM_SHARED((_N_NODES_PAD, _D), jnp.float32),
          pltpu.SemaphoreType.DMA,
          pltpu.SemaphoreType.DMA,
          pltpu.SemaphoreType.DMA,
          pltpu.SemaphoreType.DMA,
      ],
  )
  return k(emb, packed, wch, zeros_tile)


_CBLK = 1000


def _combine_mid(p, acc):
  def body(p_ref, a_ref, e_ref, ao_ref):
    e = p_ref[0] + p_ref[1]
    e_ref[...] = e
    ao_ref[...] = a_ref[...] + e

  return pl.pallas_call(
      body,
      grid=(_N_NODES // _CBLK,),
      in_specs=[
          pl.BlockSpec((2, _CBLK, _D), lambda i: (0, i, 0)),
          pl.BlockSpec((_CBLK, _D), lambda i: (i, 0)),
      ],
      out_specs=[
          pl.BlockSpec((_CBLK, _D), lambda i: (i, 0)),
          pl.BlockSpec((_CBLK, _D), lambda i: (i, 0)),
      ],
      out_shape=[
          jax.ShapeDtypeStruct((_N_NODES, _D), jnp.float32),
          jax.ShapeDtypeStruct((_N_NODES, _D), jnp.float32),
      ],
  )(p, acc)


def _combine_final(p, acc):
  def body(p_ref, a_ref, o_ref):
    o_ref[...] = (a_ref[...] + p_ref[0] + p_ref[1]) * 0.25

  return pl.pallas_call(
      body,
      grid=(_N_NODES // _CBLK,),
      in_specs=[
          pl.BlockSpec((2, _CBLK, _D), lambda i: (0, i, 0)),
          pl.BlockSpec((_CBLK, _D), lambda i: (i, 0)),
      ],
      out_specs=pl.BlockSpec((_CBLK, _D), lambda i: (i, 0)),
      out_shape=jax.ShapeDtypeStruct((_N_NODES, _D), jnp.float32),
  )(p, acc)


def _gather_sc(table, idx):
  n = idx.shape[0]
  per_w = n // _NW
  ch = 80
  nch = per_w // ch
  mesh = plsc.VectorSubcoreMesh(core_axis_name="c", subcore_axis_name="s")

  def body(t_hbm, i_hbm, o_hbm, iv, rows, sem):
    cid = jax.lax.axis_index("c")
    sid = jax.lax.axis_index("s")
    wid = sid * _NC + cid
    base = wid * per_w

    @pl.loop(0, nch)
    def _(j):
      off = base + j * ch
      pltpu.sync_copy(i_hbm.at[pl.ds(off, ch)], iv)
      pltpu.async_copy(t_hbm.at[iv], rows, sem).wait()
      pltpu.sync_copy(rows, o_hbm.at[pl.ds(off, ch)])

  k = pl.kernel(
      body,
      out_type=jax.ShapeDtypeStruct((n, _D), jnp.float32),
      mesh=mesh,
      scratch_types=[
          pltpu.VMEM((ch,), jnp.int32),
          pltpu.VMEM((ch, _D), jnp.float32),
          pltpu.SemaphoreType.DMA,
      ],
  )
  return k(table, idx)


def _gru_tc(bemb, Wt, wih0T, whh0T, bih0, bhh0, wih1T, whh1T, bih1, bhh1,
            blen4):
  nb4 = _NI * _B

  def body(e_ref, wt_ref, wih0_ref, whh0_ref, bih0_ref, bhh0_ref,
           wih1_ref, whh1_ref, bih1_ref, bhh1_ref, blen_ref, out_ref,
           h1_s, h2_s, x_s):
    t = pl.program_id(0)

    @pl.when(t == 0)
    def _():
      h1_s[...] = jnp.zeros_like(h1_s)
      h2_s[...] = jnp.zeros_like(h2_s)

    e = e_ref[0]
    for i in range(_NI):
      x_s[pl.ds(i * _B, _B), :] = jnp.dot(
          e, wt_ref[i], preferred_element_type=jnp.float32)
    x = x_s[...]

    h1p = h1_s[...]
    gi = jnp.dot(x, wih0_ref[...],
                 preferred_element_type=jnp.float32) + bih0_ref[...]
    gh = jnp.dot(h1p, whh0_ref[...],
                 preferred_element_type=jnp.float32) + bhh0_ref[...]
    r = jax.nn.sigmoid(gi[:, :_H] + gh[:, :_H])
    z = jax.nn.sigmoid(gi[:, _H:2 * _H] + gh[:, _H:2 * _H])
    n = jnp.tanh(gi[:, 2 * _H:] + r * gh[:, 2 * _H:])
    h1 = (1.0 - z) * n + z * h1p
    h1_s[...] = h1

    h2p = h2_s[...]
    gi2 = jnp.dot(h1, wih1_ref[...],
                  preferred_element_type=jnp.float32) + bih1_ref[...]
    gh2 = jnp.dot(h2p, whh1_ref[...],
                  preferred_element_type=jnp.float32) + bhh1_ref[...]
    r2 = jax.nn.sigmoid(gi2[:, :_H] + gh2[:, :_H])
    z2 = jax.nn.sigmoid(gi2[:, _H:2 * _H] + gh2[:, _H:2 * _H])
    n2 = jnp.tanh(gi2[:, 2 * _H:] + r2 * gh2[:, 2 * _H:])
    h2 = (1.0 - z2) * n2 + z2 * h2p
    h2_s[...] = h2

    sel = blen_ref[...] == t
    prev = jnp.where(t == 0, jnp.zeros_like(h2), out_ref[...])
    out_ref[...] = jnp.where(sel, h2, prev)

  return pl.pallas_call(
      body,
      grid=(_L,),
      in_specs=[
          pl.BlockSpec((1, _B, _D), lambda t: (t, 0, 0)),
          pl.BlockSpec((_NI, _D, _D), lambda t: (0, 0, 0)),
          pl.BlockSpec((_D, 3 * _H), lambda t: (0, 0)),
          pl.BlockSpec((_H, 3 * _H), lambda t: (0, 0)),
          pl.BlockSpec((1, 3 * _H), lambda t: (0, 0)),
          pl.BlockSpec((1, 3 * _H), lambda t: (0, 0)),
          pl.BlockSpec((_H, 3 * _H), lambda t: (0, 0)),
          pl.BlockSpec((_H, 3 * _H), lambda t: (0, 0)),
          pl.BlockSpec((1, 3 * _H), lambda t: (0, 0)),
          pl.BlockSpec((1, 3 * _H), lambda t: (0, 0)),
          pl.BlockSpec((nb4, _D), lambda t: (0, 0)),
      ],
      out_specs=pl.BlockSpec((nb4, _H), lambda t: (0, 0)),
      out_shape=jax.ShapeDtypeStruct((nb4, _H), jnp.float32),
      scratch_shapes=[
          pltpu.VMEM((nb4, _H), jnp.float32),
          pltpu.VMEM((nb4, _H), jnp.float32),
          pltpu.VMEM((nb4, _D), jnp.float32),
      ],
  )(bemb, Wt, wih0T, whh0T, bih0, bhh0, wih1T, whh1T, bih1, bhh1, blen4)


def _final_tc(E, items, merge_w):
  n_items = items.shape[0]

  def body(e_ref, it_ref, mw_ref, o_ref):
    Ei = [e_ref[pl.ds(i * _B, _B), :] for i in range(_NI)]
    g = [[jnp.sum(Ei[i] * Ei[j], axis=1, keepdims=True)
          for j in range(_NI)] for i in range(_NI)]
    c = None
    for i in range(_NI):
      m = jnp.maximum(jnp.maximum(g[i][0], g[i][1]),
                      jnp.maximum(g[i][2], g[i][3]))
      ex = [jnp.exp(g[i][j] - m) for j in range(_NI)]
      s = ex[0] + ex[1] + ex[2] + ex[3]
      num = (ex[0] * mw_ref[0, 0] + ex[1] * mw_ref[0, 1]
             + ex[2] * mw_ref[0, 2] + ex[3] * mw_ref[0, 3])
      ui = num / s
      c = ui * Ei[i] if c is None else c + ui * Ei[i]
    o_ref[...] = jax.lax.dot_general(
        c, it_ref[...], (((1,), (1,)), ((), ())),
        preferred_element_type=jnp.float32)

  return pl.pallas_call(
      body,
      grid=(1,),
      in_specs=[
          pl.BlockSpec((_NI * _B, _H), lambda i: (0, 0)),
          pl.BlockSpec((n_items, _D), lambda i: (0, 0)),
          pl.BlockSpec(memory_space=pltpu.SMEM),
      ],
      out_specs=pl.BlockSpec((_B, n_items), lambda i: (0, 0)),
      out_shape=jax.ShapeDtypeStruct((_B, n_items), jnp.float32),
  )(E, items, merge_w)


def kernel(bseq, bseq_len, edge_index, edge_weight, emb_basket, emb_item,
           W_bseq, merge_W, Wih0, Whh0, bih0, bhh0, Wih1, Whh1, bih1, bhh1):
  f32 = jnp.float32
  e0 = jnp.concatenate([emb_basket, emb_item], axis=0)

  src = edge_index[0].astype(jnp.int32)
  dst = edge_index[1].astype(jnp.int32)
  w = edge_weight.astype(f32)
  ne = src.shape[0]
  gran = 2 * _EDGE_CHUNK
  per_w = ((ne + _NW - 1) // _NW + gran - 1) // gran * gran
  pad = per_w * _NW - ne
  src_p = jnp.concatenate([src, jnp.zeros((pad,), jnp.int32)])
  dst_p = jnp.concatenate([dst, jnp.zeros((pad,), jnp.int32)])
  w_p = jnp.concatenate([w, jnp.zeros((pad,), f32)])
  n_chunks = per_w // _EDGE_CHUNK
  packed = jnp.stack([
      src_p.reshape(_NW * n_chunks, _EDGE_CHUNK),
      dst_p.reshape(_NW * n_chunks, _EDGE_CHUNK),
  ], axis=1)
  wch = w_p.reshape(_NW * n_chunks, _EDGE_CHUNK)
  zeros_tile = jnp.zeros((_ROWS_PER_TILE, _D), f32)

  emb = e0
  acc = e0
  light = None
  for layer in range(3):
    p = _gcn_layer_sc(emb, packed, wch, zeros_tile, n_chunks)
    if layer < 2:
      emb, acc = _combine_mid(p, acc)
    else:
      light = _combine_final(p, acc)

  idx_t = jnp.transpose(bseq).reshape(-1).astype(jnp.int32)
  bemb = _gather_sc(light, idx_t).reshape(_L, _B, _D)

  Wt = jnp.transpose(W_bseq.reshape(_D, _NI, _D), (1, 2, 0))

  blen = jnp.tile(bseq_len.astype(jnp.int32) - 1, _NI)
  blen4 = jnp.broadcast_to(blen[:, None], (_NI * _B, _H))

  E = _gru_tc(bemb, Wt,
              jnp.transpose(Wih0), jnp.transpose(Whh0),
              bih0.reshape(1, -1), bhh0.reshape(1, -1),
              jnp.transpose(Wih1), jnp.transpose(Whh1),
              bih1.reshape(1, -1), bhh1.reshape(1, -1),
              blen4)

  items = light[_NUM_BASKETS:]
  return _final_tc(E, items, merge_W.astype(f32))
